# Initial kernel scaffold; baseline (speedup 1.0000x reference)
import jax
import jax.numpy as jnp
from jax.experimental import pallas as pl


def kernel(seasonal, trend, W, b, ln_scale, ln_bias):
    raise NotImplementedError("write your pallas kernel here")


# speedup vs baseline: 1.8488x; 1.8488x over previous
import jax
import jax.numpy as jnp
from jax.experimental import pallas as pl

TOPK = 16
EPS = 1e-6


def _encoder_tail_kernel(corr_ref, seas_ref, w_ref, b_ref, scale_ref, bias_ref,
                         out_ref):
    c = corr_ref[0]
    L, C = c.shape
    mag = jnp.abs(c)
    iota = jax.lax.broadcasted_iota(jnp.int32, (L, C), 0)

    def body(_, carry):
        work, maskacc = carry
        m = jnp.max(work, axis=0)
        eq = work == m[None, :]
        idx = jnp.where(eq, iota, L)
        imin = jnp.min(idx, axis=0)
        sel = eq & (iota == imin[None, :])
        maskacc = jnp.where(sel, 1.0, maskacc)
        work = jnp.where(sel, -1.0, work)
        return work, maskacc

    _, mask = jax.lax.fori_loop(
        0, TOPK, body, (mag, jnp.zeros((L, C), jnp.float32)))

    masked = c * mask
    ff = jax.lax.dot_general(
        masked, w_ref[...],
        (((1,), (0,)), ((), ())),
        preferred_element_type=jnp.float32,
        precision=jax.lax.Precision.HIGHEST,
    ) + b_ref[0][None, :]
    x = seas_ref[0] + ff
    mean = jnp.mean(x, axis=1, keepdims=True)
    xc = x - mean
    var = jnp.mean(xc * xc, axis=1, keepdims=True)
    normed = xc * jax.lax.rsqrt(var + EPS)
    out_ref[0] = normed * scale_ref[0][None, :] + bias_ref[0][None, :]


def kernel(seasonal, trend, W, b, ln_scale, ln_bias):
    B, L, C = seasonal.shape
    D = W.shape[1]
    X = jnp.fft.rfft(seasonal, axis=1)
    P = X * jnp.conj(X)
    corr = jnp.fft.irfft(P, n=L, axis=1)

    seasonal_out = pl.pallas_call(
        _encoder_tail_kernel,
        grid=(B,),
        in_specs=[
            pl.BlockSpec((1, L, C), lambda i: (i, 0, 0)),
            pl.BlockSpec((1, L, C), lambda i: (i, 0, 0)),
            pl.BlockSpec((C, D), lambda i: (0, 0)),
            pl.BlockSpec((1, D), lambda i: (0, 0)),
            pl.BlockSpec((1, D), lambda i: (0, 0)),
            pl.BlockSpec((1, D), lambda i: (0, 0)),
        ],
        out_specs=pl.BlockSpec((1, L, D), lambda i: (i, 0, 0)),
        out_shape=jax.ShapeDtypeStruct((B, L, D), jnp.float32),
    )(corr, seasonal, W, b.reshape(1, D), ln_scale.reshape(1, D),
      ln_bias.reshape(1, D))
    return (seasonal_out, trend)


# speedup vs baseline: 2.0951x; 1.1332x over previous
import jax
import jax.numpy as jnp
from jax.experimental import pallas as pl

TOPK = 16
EPS = 1e-6


def _encoder_tail_kernel(corr_ref, seas_ref, wk_ref, s1_ref, s2_ref, b_ref,
                         scale_ref, bias_ref, out_ref):
    BL, C = corr_ref.shape
    BC = wk_ref.shape[0]
    B = BC // C
    L = BL // B
    hi = jax.lax.Precision.HIGHEST

    c = jnp.concatenate(
        [corr_ref[pl.ds(i * L, L), :] for i in range(B)], axis=1)
    s = jnp.concatenate(
        [seas_ref[pl.ds(i * L, L), :] for i in range(B)], axis=1)

    mag = jnp.abs(c)
    iota = jax.lax.broadcasted_iota(jnp.int32, (L, BC), 0)
    work0 = jnp.where(iota == 0, -1.0, mag)

    def body(_, work):
        m = jnp.max(work, axis=0)
        idx = jnp.where(work == m[None, :], iota, L)
        imin = jnp.min(idx, axis=0)
        return jnp.where(iota == imin[None, :], -1.0, work)

    work = jax.lax.fori_loop(0, TOPK - 1, body, work0)

    masked = jnp.where(work < 0, c, 0.0)
    ff = jax.lax.dot_general(
        masked, wk_ref[...], (((1,), (0,)), ((), ())),
        preferred_element_type=jnp.float32, precision=hi) + b_ref[0][None, :]
    x = s + ff

    def seg_mean(v):
        t = jax.lax.dot_general(v, s1_ref[...], (((1,), (0,)), ((), ())),
                                preferred_element_type=jnp.float32,
                                precision=hi)
        return jax.lax.dot_general(t, s2_ref[...], (((1,), (0,)), ((), ())),
                                   preferred_element_type=jnp.float32,
                                   precision=hi)

    mean = seg_mean(x)
    xc = x - mean
    var = seg_mean(xc * xc)
    normed = xc * jax.lax.rsqrt(var + EPS)
    o = normed * scale_ref[0][None, :] + bias_ref[0][None, :]
    for i in range(B):
        out_ref[pl.ds(i * L, L), :] = o[:, i * C:(i + 1) * C]


def kernel(seasonal, trend, W, b, ln_scale, ln_bias):
    B, L, C = seasonal.shape
    D = W.shape[1]
    X = jnp.fft.rfft(seasonal, axis=1)
    P = X * jnp.conj(X)
    corr = jnp.fft.irfft(P, n=L, axis=1)

    eye = jnp.eye(B, dtype=jnp.float32)
    wk = jnp.kron(eye, W)
    s1 = jnp.kron(eye, jnp.full((D, 1), 1.0 / D))
    s2 = jnp.kron(eye, jnp.ones((1, D), jnp.float32))
    bt = jnp.tile(b, B).reshape(1, B * D)
    st = jnp.tile(ln_scale, B).reshape(1, B * D)
    bst = jnp.tile(ln_bias, B).reshape(1, B * D)

    out2 = pl.pallas_call(
        _encoder_tail_kernel,
        out_shape=jax.ShapeDtypeStruct((B * L, D), jnp.float32),
    )(corr.reshape(B * L, C), seasonal.reshape(B * L, C), wk, s1, s2,
      bt, st, bst)
    return (out2.reshape(B, L, D), trend)


# speedup vs baseline: 2.1657x; 1.0337x over previous
import jax
import jax.numpy as jnp
from jax.experimental import pallas as pl

TOPK = 16
EPS = 1e-6


def _encoder_tail_kernel(corr_ref, seas_ref, wk_ref, k1_ref, m_ref, b_ref,
                         scale_ref, bias_ref, out_ref):
    BL, C = corr_ref.shape
    BC = wk_ref.shape[0]
    B = BC // C
    L = BL // B
    hi = jax.lax.Precision.HIGHEST

    c = jnp.concatenate(
        [corr_ref[pl.ds(i * L, L), :] for i in range(B)], axis=1)
    s = jnp.concatenate(
        [seas_ref[pl.ds(i * L, L), :] for i in range(B)], axis=1)

    mag = jnp.abs(c)
    iota = jax.lax.broadcasted_iota(jnp.int32, (L, BC), 0)
    work0 = jnp.where(iota == 0, -1.0, mag)

    def body(_, work):
        m = jnp.max(work, axis=0)
        idx = jnp.where(work == m[None, :], iota, L)
        imin = jnp.min(idx, axis=0)
        return jnp.where(iota == imin[None, :], -1.0, work)

    work = jax.lax.fori_loop(0, TOPK - 1, body, work0)

    masked = jnp.where(work < 0, c, 0.0)
    ff = jax.lax.dot_general(
        masked, wk_ref[...], (((1,), (0,)), ((), ())),
        preferred_element_type=jnp.float32, precision=hi) + b_ref[0][None, :]
    x = s + ff

    xc = jax.lax.dot_general(x, k1_ref[...], (((1,), (0,)), ((), ())),
                             preferred_element_type=jnp.float32, precision=hi)
    var = jax.lax.dot_general(xc * xc, m_ref[...], (((1,), (0,)), ((), ())),
                              preferred_element_type=jnp.float32, precision=hi)
    normed = xc * jax.lax.rsqrt(var + EPS)
    o = normed * scale_ref[0][None, :] + bias_ref[0][None, :]
    for i in range(B):
        out_ref[pl.ds(i * L, L), :] = o[:, i * C:(i + 1) * C]


def kernel(seasonal, trend, W, b, ln_scale, ln_bias):
    B, L, C = seasonal.shape
    D = W.shape[1]
    X = jnp.fft.rfft(seasonal, axis=1)
    P = X * jnp.conj(X)
    corr = jnp.fft.irfft(P, n=L, axis=1)

    eye = jnp.eye(B, dtype=jnp.float32)
    wk = jnp.kron(eye, W)
    mseg = jnp.kron(eye, jnp.full((D, D), 1.0 / D))
    k1 = jnp.eye(B * D, dtype=jnp.float32) - mseg
    bt = jnp.tile(b, B).reshape(1, B * D)
    st = jnp.tile(ln_scale, B).reshape(1, B * D)
    bst = jnp.tile(ln_bias, B).reshape(1, B * D)

    out2 = pl.pallas_call(
        _encoder_tail_kernel,
        out_shape=jax.ShapeDtypeStruct((B * L, D), jnp.float32),
    )(corr.reshape(B * L, C), seasonal.reshape(B * L, C), wk, k1, mseg,
      bt, st, bst)
    return (out2.reshape(B, L, D), trend)


# speedup vs baseline: 2.3698x; 1.0942x over previous
import jax
import jax.numpy as jnp
from jax.experimental import pallas as pl

TOPK = 16
EPS = 1e-6


def _encoder_tail_kernel(corr_ref, seas_ref, wk_ref, b_ref, scale_ref,
                         bias_ref, out_ref):
    BC, L = corr_ref.shape
    C = out_ref.shape[1]
    B = BC // C
    hi = jax.lax.Precision.HIGHEST

    c = corr_ref[...]
    mag = jnp.abs(c)
    iota = jax.lax.broadcasted_iota(jnp.int32, (BC, L), 1)
    work0 = jnp.where(iota == 0, -1.0, mag)

    def body(_, work):
        m = jnp.max(work, axis=1)
        idx = jnp.where(work == m[:, None], iota, L)
        imin = jnp.min(idx, axis=1)
        return jnp.where(iota == imin[:, None], -1.0, work)

    work = jax.lax.fori_loop(0, TOPK - 1, body, work0)

    masked = jnp.where(work < 0, c, 0.0)
    ff = jax.lax.dot_general(
        wk_ref[...], masked, (((0,), (0,)), ((), ())),
        preferred_element_type=jnp.float32, precision=hi)
    x = seas_ref[...] + ff + b_ref[...]

    xr = x.reshape(B, C, L)
    mean = jnp.mean(xr, axis=1, keepdims=True)
    xc = xr - mean
    var = jnp.mean(xc * xc, axis=1, keepdims=True)
    normed = xc * jax.lax.rsqrt(var + EPS)
    o = normed.reshape(BC, L) * scale_ref[...] + bias_ref[...]
    for i in range(B):
        out_ref[pl.ds(i * L, L), :] = o[i * C:(i + 1) * C, :].T


def kernel(seasonal, trend, W, b, ln_scale, ln_bias):
    B, L, C = seasonal.shape
    D = W.shape[1]
    X = jnp.fft.rfft(seasonal, axis=1)
    P = X * jnp.conj(X)
    corr = jnp.fft.irfft(P, n=L, axis=1)

    corr_t = jnp.transpose(corr, (0, 2, 1)).reshape(B * C, L)
    seas_t = jnp.transpose(seasonal, (0, 2, 1)).reshape(B * C, L)

    wk = jnp.kron(jnp.eye(B, dtype=jnp.float32), W)
    bt = jnp.tile(b, B).reshape(B * D, 1)
    st = jnp.tile(ln_scale, B).reshape(B * D, 1)
    bst = jnp.tile(ln_bias, B).reshape(B * D, 1)

    out2 = pl.pallas_call(
        _encoder_tail_kernel,
        out_shape=jax.ShapeDtypeStruct((B * L, D), jnp.float32),
    )(corr_t, seas_t, wk, bt, st, bst)
    return (out2.reshape(B, L, D), trend)
